# Initial kernel scaffold; baseline (speedup 1.0000x reference)
import jax
import jax.numpy as jnp
from jax.experimental import pallas as pl


def kernel(x, embeddings):
    raise NotImplementedError("write your pallas kernel here")


# speedup vs baseline: 3.9885x; 3.9885x over previous
import functools

import jax
import jax.numpy as jnp
from jax import lax
from jax.experimental import pallas as pl
from jax.experimental.pallas import tpu as pltpu
from jax.experimental.pallas import tpu_sc as plsc

V_DIM = 100000
EMB_DIM = 64
BATCH = 4096
CTX = 50

NC = 2
NS = 16
NW = NC * NS
BPW = BATCH // NW
CH = 2
NCH = BPW // CH
IDX_PAD = 104
NREG = EMB_DIM // 16


def _cbow_kernel(idx_hbm, table_hbm, out_hbm, idx_v, rows_v, out_v,
                 sem0, sem1):
    wid = lax.axis_index("s") * NC + lax.axis_index("c")

    pltpu.sync_copy(idx_hbm.at[wid], idx_v)

    sems = (sem0, sem1)

    def gather_descr(c, b, sem):
        return pltpu.make_async_copy(
            table_hbm.at[idx_v.at[c]], rows_v.at[b], sem)

    def accumulate(c, b):
        def body(j, carry):
            acc = list(carry)
            for r in range(CH):
                rr = r * CTX + j
                for d in range(NREG):
                    acc[r * NREG + d] = acc[r * NREG + d] + rows_v[
                        b, rr, pl.ds(d * 16, 16)]
            return tuple(acc)

        zero = jnp.zeros((16,), jnp.float32)
        acc = lax.fori_loop(0, CTX, body, (zero,) * (CH * NREG))
        scale = jnp.float32(1.0 / CTX)
        for r in range(CH):
            for d in range(NREG):
                out_v[c * CH + r, pl.ds(d * 16, 16)] = (
                    acc[r * NREG + d] * scale)

    gather_descr(0, 0, sems[0]).start()

    def outer(cc, carry):
        for b in range(2):
            c = cc * 2 + b
            nb = 1 - b

            @pl.when(c + 1 < NCH)
            def _():
                gather_descr(c + 1, nb, sems[nb]).start()

            gather_descr(c, b, sems[b]).wait()
            accumulate(c, b)
        return carry

    lax.fori_loop(0, NCH // 2, outer, 0)

    pltpu.sync_copy(out_v, out_hbm.at[pl.ds(wid * BPW, BPW)])


@jax.jit
def kernel(x, embeddings):
    idx = x.astype(jnp.int32).reshape(NW, NCH, CH * CTX)
    idx = jnp.pad(idx, ((0, 0), (0, 0), (0, IDX_PAD - CH * CTX)))

    mesh = plsc.VectorSubcoreMesh(core_axis_name="c", subcore_axis_name="s")
    run = pl.kernel(
        _cbow_kernel,
        mesh=mesh,
        out_type=jax.ShapeDtypeStruct((BATCH, EMB_DIM), jnp.float32),
        scratch_types=[
            pltpu.VMEM((NCH, IDX_PAD), jnp.int32),
            pltpu.VMEM((2, IDX_PAD, EMB_DIM), jnp.float32),
            pltpu.VMEM((BPW, EMB_DIM), jnp.float32),
            pltpu.SemaphoreType.DMA,
            pltpu.SemaphoreType.DMA,
        ],
        compiler_params=pltpu.CompilerParams(use_tc_tiling_on_sc=False),
    )
    return run(idx, embeddings)
